# Initial kernel scaffold; baseline (speedup 1.0000x reference)
import jax
import jax.numpy as jnp
from jax.experimental import pallas as pl


def kernel(x, edge_index, edge_attr, Wl, bl, Wr, Wf0, bf0, Wout, bout):
    raise NotImplementedError("write your pallas kernel here")


# speedup vs baseline: 2.7731x; 2.7731x over previous
import functools

import jax
import jax.numpy as jnp
from jax import lax
from jax.experimental import pallas as pl
from jax.experimental.pallas import tpu as pltpu
from jax.experimental.pallas import tpu_sc as plsc

NC = 2
NS = 16
NW = NC * NS
K = 80


def _leaky(v):
    return jnp.where(v >= 0, v, 0.1 * v)


def _make_s1(N, Np, D, E):
    epw = E // NW
    nchunk = epw // K
    rps = Np // NS
    nio = rps // K
    mesh = plsc.VectorSubcoreMesh(core_axis_name="c", subcore_axis_name="s")

    @functools.partial(
        pl.kernel,
        mesh=mesh,
        out_type=[
            jax.ShapeDtypeStruct((NC, Np, D), jnp.float32),
            jax.ShapeDtypeStruct((NC, Np, 16), jnp.float32),
        ],
        scratch_types=[
            pltpu.VMEM((K,), jnp.int32),
            pltpu.VMEM((K,), jnp.int32),
            pltpu.VMEM((K, D), jnp.float32),
            pltpu.VMEM((K, 16), jnp.float32),
            pltpu.VMEM((K, 16), jnp.float32),
            pltpu.VMEM_SHARED((Np, D), jnp.float32),
            pltpu.VMEM_SHARED((Np, 16), jnp.float32),
        ],
        compiler_params=pltpu.CompilerParams(use_tc_tiling_on_sc=False),
    )
    def s1(x_hbm, src_hbm, dst_hbm, zx_hbm, zc_hbm, ones_hbm,
           sums_out, cnt_out, idx_s, idx_d, rows, ones_v, c16, acc, acc_cnt):
        c = lax.axis_index("c")
        s = lax.axis_index("s")
        wid = s * NC + c
        pltpu.sync_copy(zx_hbm, rows)
        pltpu.sync_copy(zc_hbm, c16)
        pltpu.sync_copy(ones_hbm, ones_v)

        def zchunk(j, carry):
            pltpu.sync_copy(rows, acc.at[pl.ds(s * rps + j * K, K)])
            pltpu.sync_copy(c16, acc_cnt.at[pl.ds(s * rps + j * K, K)])
            return carry

        lax.fori_loop(0, nio, zchunk, 0)
        plsc.subcore_barrier()

        base = wid * epw

        def chunk(i, carry):
            off = base + i * K
            pltpu.sync_copy(src_hbm.at[pl.ds(off, K)], idx_s)
            pltpu.sync_copy(dst_hbm.at[pl.ds(off, K)], idx_d)
            pltpu.sync_copy(x_hbm.at[idx_s], rows)
            pltpu.sync_copy(rows, acc.at[idx_d], add=True)
            pltpu.sync_copy(ones_v, acc_cnt.at[idx_d], add=True)
            return carry

        lax.fori_loop(0, nchunk, chunk, 0)
        plsc.subcore_barrier()

        def ochunk(j, carry):
            r0 = s * rps + j * K
            pltpu.sync_copy(acc.at[pl.ds(r0, K)], rows)
            pltpu.sync_copy(rows, sums_out.at[c, pl.ds(r0, K)])
            pltpu.sync_copy(acc_cnt.at[pl.ds(r0, K)], c16)
            pltpu.sync_copy(c16, cnt_out.at[c, pl.ds(r0, K)])
            return carry

        lax.fori_loop(0, nio, ochunk, 0)

    return s1


def _t1_body(sp_ref, cp_ref, x_ref, wl_ref, bl_ref, wr_ref, wf_ref,
             a_ref, b_ref, *, N, D):
    sums = sp_ref[0, :N] + sp_ref[1, :N]
    cnt = cp_ref[0, :N, 0:1] + cp_ref[1, :N, 0:1]
    agg = sums / jnp.maximum(cnt, 1.0)
    h0 = (jnp.dot(agg, wl_ref[...], preferred_element_type=jnp.float32)
          + jnp.dot(x_ref[...], wr_ref[...], preferred_element_type=jnp.float32)
          + bl_ref[...])
    mu = jnp.mean(h0, axis=0, keepdims=True)
    xc = h0 - mu
    denom = jnp.sqrt(1e-5 + jnp.sum(xc * xc) / N)
    h = _leaky(xc / denom)
    a_ref[...] = jnp.dot(h, wf_ref[:D], preferred_element_type=jnp.float32)
    b_ref[...] = jnp.dot(h, wf_ref[D:], preferred_element_type=jnp.float32)


def _make_s2(N, F1, E):
    epw = E // NW
    nchunk = epw // K
    mesh = plsc.VectorSubcoreMesh(core_axis_name="c", subcore_axis_name="s")

    @functools.partial(
        pl.kernel,
        mesh=mesh,
        out_type=[
            jax.ShapeDtypeStruct((E, F1), jnp.float32),
            jax.ShapeDtypeStruct((E, F1), jnp.float32),
        ],
        scratch_types=[
            pltpu.VMEM((K,), jnp.int32),
            pltpu.VMEM((K,), jnp.int32),
            pltpu.VMEM((K, F1), jnp.float32),
            pltpu.VMEM((K, F1), jnp.float32),
        ],
        compiler_params=pltpu.CompilerParams(use_tc_tiling_on_sc=False),
    )
    def s2(a_hbm, b_hbm, src_hbm, dst_hbm, as_out, bd_out,
           idx_s, idx_d, buf_a, buf_b):
        c = lax.axis_index("c")
        s = lax.axis_index("s")
        wid = s * NC + c
        base = wid * epw

        def chunk(i, carry):
            off = base + i * K
            pltpu.sync_copy(src_hbm.at[pl.ds(off, K)], idx_s)
            pltpu.sync_copy(dst_hbm.at[pl.ds(off, K)], idx_d)
            pltpu.sync_copy(a_hbm.at[idx_s], buf_a)
            pltpu.sync_copy(b_hbm.at[idx_d], buf_b)
            pltpu.sync_copy(buf_a, as_out.at[pl.ds(off, K)])
            pltpu.sync_copy(buf_b, bd_out.at[pl.ds(off, K)])
            return carry

        lax.fori_loop(0, nchunk, chunk, 0)

    return s2


def _t2_body(as_ref, bd_ref, bf_ref, wo_ref, bo_ref, out_ref):
    z = as_ref[...] + bd_ref[...] + bf_ref[...]
    e = _leaky(z)
    out_ref[...] = (jnp.dot(e, wo_ref[...], preferred_element_type=jnp.float32)
                    + bo_ref[...])


def kernel(x, edge_index, edge_attr, Wl, bl, Wr, Wf0, bf0, Wout, bout):
    del edge_attr
    N, D = x.shape
    E = edge_index.shape[1]
    F1 = Wf0.shape[1]
    OUT = Wout.shape[1]
    src = edge_index[0]
    dst = edge_index[1]

    Np = ((N + (NS * K) - 1) // (NS * K)) * (NS * K)
    zx = jnp.zeros((K, D), jnp.float32)
    zc = jnp.zeros((K, 16), jnp.float32)
    ones = jnp.ones((K, 16), jnp.float32)

    sums_p, cnt_p = _make_s1(N, Np, D, E)(x, src, dst, zx, zc, ones)

    a_tab, b_tab = pl.pallas_call(
        functools.partial(_t1_body, N=N, D=D),
        out_shape=[
            jax.ShapeDtypeStruct((N, F1), jnp.float32),
            jax.ShapeDtypeStruct((N, F1), jnp.float32),
        ],
    )(sums_p, cnt_p, x, Wl, bl.reshape(1, D), Wr, Wf0)

    a_e, b_e = _make_s2(N, F1, E)(a_tab, b_tab, src, dst)

    BE = 8000
    out = pl.pallas_call(
        _t2_body,
        grid=(E // BE,),
        in_specs=[
            pl.BlockSpec((BE, F1), lambda i: (i, 0)),
            pl.BlockSpec((BE, F1), lambda i: (i, 0)),
            pl.BlockSpec((1, F1), lambda i: (0, 0)),
            pl.BlockSpec((F1, OUT), lambda i: (0, 0)),
            pl.BlockSpec((1, OUT), lambda i: (0, 0)),
        ],
        out_specs=pl.BlockSpec((BE, OUT), lambda i: (i, 0)),
        out_shape=jax.ShapeDtypeStruct((E, OUT), jnp.float32),
    )(a_e, b_e, bf0.reshape(1, F1), Wout, bout.reshape(1, OUT))

    return out
